# Initial kernel scaffold; baseline (speedup 1.0000x reference)
import jax
import jax.numpy as jnp
from jax.experimental import pallas as pl


def kernel(x, edge_index, W_l, b_l, W_r):
    raise NotImplementedError("write your pallas kernel here")


# speedup vs baseline: 6.4058x; 6.4058x over previous
import functools

import jax
import jax.numpy as jnp
from jax import lax
from jax.experimental import pallas as pl
from jax.experimental.pallas import tpu as pltpu
from jax.experimental.pallas import tpu_sc as plsc

N = 10000
NPAD = 10240
E = 320000
D = 128
ROWW = D + 16
NC, NS = 2, 16
NW = NC * NS
CH = 128
NCHUNKS = E // CH
RPT = NPAD // NS


def _sc_body(xpad_hbm, src_hbm, dst_hbm, zeros_hbm, out_hbm,
             sidx, didx, rows, acc, sem):
    c = lax.axis_index("c")
    s = lax.axis_index("s")
    wid = c * NS + s

    row0 = pl.multiple_of(s * RPT, 8)
    pltpu.sync_copy(zeros_hbm, acc.at[pl.ds(row0, RPT)])
    plsc.subcore_barrier()

    nchunk = NCHUNKS // NW + (wid < NCHUNKS % NW).astype(jnp.int32)

    def step(i, carry):
        off = (wid + i * NW) * CH
        pltpu.sync_copy(src_hbm.at[pl.ds(off, CH)], sidx)
        pltpu.sync_copy(dst_hbm.at[pl.ds(off, CH)], didx)
        pltpu.async_copy(xpad_hbm.at[sidx], rows, sem).wait()
        pltpu.sync_copy(rows, acc.at[didx], add=True)
        return carry

    lax.fori_loop(0, nchunk, step, 0)
    plsc.subcore_barrier()

    pltpu.sync_copy(acc.at[pl.ds(row0, RPT)],
                    out_hbm.at[pl.ds(pl.multiple_of(c * NPAD + s * RPT, 8),
                                     RPT)])


_sc_aggregate = functools.partial(
    pl.kernel,
    out_type=jax.ShapeDtypeStruct((NC * NPAD, ROWW), jnp.float32),
    mesh=plsc.VectorSubcoreMesh(core_axis_name="c", subcore_axis_name="s",
                                num_cores=NC, num_subcores=NS),
    scratch_types=[
        pltpu.VMEM((CH,), jnp.int32),
        pltpu.VMEM((CH,), jnp.int32),
        pltpu.VMEM((CH, ROWW), jnp.float32),
        pltpu.VMEM_SHARED((NPAD, ROWW), jnp.float32),
        pltpu.SemaphoreType.DMA,
    ],
    compiler_params=pltpu.CompilerParams(use_tc_tiling_on_sc=False),
)(_sc_body)


def _tc_body(p_ref, x_ref, wl_ref, wr_ref, b_ref, o_ref):
    p = p_ref[0] + p_ref[1]
    cnt = p[:, D:D + 1]
    mean = p[:, :D] / jnp.maximum(cnt, 1.0)
    o_ref[...] = (
        jnp.dot(mean, wl_ref[...], preferred_element_type=jnp.float32)
        + jnp.dot(x_ref[...], wr_ref[...], preferred_element_type=jnp.float32)
        + b_ref[...]
    )


def _tc_combine(partial, x, wl_t, wr_t, b2):
    R = 1000
    grid = (N // R,)
    return pl.pallas_call(
        _tc_body,
        grid=grid,
        in_specs=[
            pl.BlockSpec((NC, R, ROWW), lambda i: (0, i, 0)),
            pl.BlockSpec((R, D), lambda i: (i, 0)),
            pl.BlockSpec((D, D), lambda i: (0, 0)),
            pl.BlockSpec((D, D), lambda i: (0, 0)),
            pl.BlockSpec((1, D), lambda i: (0, 0)),
        ],
        out_specs=pl.BlockSpec((R, D), lambda i: (i, 0)),
        out_shape=jax.ShapeDtypeStruct((N, D), jnp.float32),
    )(partial, x, wl_t, wr_t, b2)


def kernel(x, edge_index, W_l, b_l, W_r):
    src = edge_index[0]
    dst = edge_index[1]
    xpad = jnp.concatenate([x, jnp.ones((N, ROWW - D), jnp.float32)], axis=1)
    zeros = jnp.zeros((RPT, ROWW), jnp.float32)
    partial = _sc_aggregate(xpad, src, dst, zeros)
    partial = partial.reshape(NC, NPAD, ROWW)[:, :N]
    return _tc_combine(partial, x, W_l.T, W_r.T, b_l.reshape(1, D))


# speedup vs baseline: 8.3395x; 1.3019x over previous
import functools

import jax
import jax.numpy as jnp
from jax import lax
from jax.experimental import pallas as pl
from jax.experimental.pallas import tpu as pltpu
from jax.experimental.pallas import tpu_sc as plsc

N = 10000
NPAD = 10240
E = 320000
D = 128
ROWW = D + 16
NC, NS = 2, 16
NW = NC * NS
CH = 128
NCHUNKS = E // CH
SLOTS = 80
ZPT = NPAD // NS
DPT = N // NS


def _sc_body(xpad_hbm, src_hbm, dst_hbm, zeros_hbm, out_hbm,
             sidx0, didx0, sidx1, didx1, rows0, rows1, acc,
             si0, si1, sg0, sg1):
    c = lax.axis_index("c")
    s = lax.axis_index("s")
    wid = c * NS + s

    pltpu.sync_copy(zeros_hbm, acc.at[pl.ds(pl.multiple_of(s * ZPT, 8), ZPT)])

    def off(t):
        return jnp.minimum(wid + t * NW, NCHUNKS - 1) * CH

    def idx_start(t, sbuf, dbuf, sem):
        o = off(t)
        pltpu.async_copy(src_hbm.at[pl.ds(o, CH)], sbuf, sem)
        pltpu.async_copy(dst_hbm.at[pl.ds(o, CH)], dbuf, sem)

    def idx_wait(sbuf, dbuf, sem):
        pltpu.make_async_copy(src_hbm.at[pl.ds(0, CH)], sbuf, sem).wait()
        pltpu.make_async_copy(dst_hbm.at[pl.ds(0, CH)], dbuf, sem).wait()

    def gath(sbuf, rbuf, sem):
        pltpu.async_copy(xpad_hbm.at[sbuf], rbuf, sem)

    def gwait(sbuf, rbuf, sem):
        pltpu.make_async_copy(xpad_hbm.at[sbuf], rbuf, sem).wait()

    def scat(t, dbuf, rbuf):
        @pl.when(wid + t * NW < NCHUNKS)
        def _():
            pltpu.sync_copy(rbuf, acc.at[dbuf], add=True)

    idx_start(0, sidx0, didx0, si0)
    idx_start(1, sidx1, didx1, si1)
    idx_wait(sidx0, didx0, si0)
    gath(sidx0, rows0, sg0)

    def body(i, carry):
        a = 2 * i
        b = a + 1
        idx_wait(sidx1, didx1, si1)
        gwait(sidx0, rows0, sg0)
        scat(a, didx0, rows0)
        idx_start(a + 2, sidx0, didx0, si0)
        gath(sidx1, rows1, sg1)
        gwait(sidx1, rows1, sg1)
        scat(b, didx1, rows1)
        idx_wait(sidx0, didx0, si0)
        gath(sidx0, rows0, sg0)
        idx_start(b + 2, sidx1, didx1, si1)
        return carry

    lax.fori_loop(0, SLOTS // 2, body, 0)
    idx_wait(sidx1, didx1, si1)
    gwait(sidx0, rows0, sg0)

    plsc.subcore_barrier()
    pltpu.sync_copy(acc.at[pl.ds(s * DPT, DPT)],
                    out_hbm.at[pl.ds(c * N + s * DPT, DPT)])


_sc_aggregate = functools.partial(
    pl.kernel,
    out_type=jax.ShapeDtypeStruct((NC * N, ROWW), jnp.float32),
    mesh=plsc.VectorSubcoreMesh(core_axis_name="c", subcore_axis_name="s",
                                num_cores=NC, num_subcores=NS),
    scratch_types=[
        pltpu.VMEM((CH,), jnp.int32),
        pltpu.VMEM((CH,), jnp.int32),
        pltpu.VMEM((CH,), jnp.int32),
        pltpu.VMEM((CH,), jnp.int32),
        pltpu.VMEM((CH, ROWW), jnp.float32),
        pltpu.VMEM((CH, ROWW), jnp.float32),
        pltpu.VMEM_SHARED((NPAD, ROWW), jnp.float32),
        pltpu.SemaphoreType.DMA,
        pltpu.SemaphoreType.DMA,
        pltpu.SemaphoreType.DMA,
        pltpu.SemaphoreType.DMA,
    ],
    compiler_params=pltpu.CompilerParams(use_tc_tiling_on_sc=False),
)(_sc_body)


def _tc_body(p_ref, x_ref, wl_ref, wr_ref, b_ref, o_ref):
    p = p_ref[0] + p_ref[1]
    cnt = p[:, D:D + 1]
    mean = p[:, :D] / jnp.maximum(cnt, 1.0)
    o_ref[...] = (
        jnp.dot(mean, wl_ref[...], preferred_element_type=jnp.float32)
        + jnp.dot(x_ref[...], wr_ref[...], preferred_element_type=jnp.float32)
        + b_ref[...]
    )


def _tc_combine(partial, x, wl_t, wr_t, b2):
    R = 1000
    grid = (N // R,)
    return pl.pallas_call(
        _tc_body,
        grid=grid,
        in_specs=[
            pl.BlockSpec((NC, R, ROWW), lambda i: (0, i, 0)),
            pl.BlockSpec((R, D), lambda i: (i, 0)),
            pl.BlockSpec((D, D), lambda i: (0, 0)),
            pl.BlockSpec((D, D), lambda i: (0, 0)),
            pl.BlockSpec((1, D), lambda i: (0, 0)),
        ],
        out_specs=pl.BlockSpec((R, D), lambda i: (i, 0)),
        out_shape=jax.ShapeDtypeStruct((N, D), jnp.float32),
    )(partial, x, wl_t, wr_t, b2)


def kernel(x, edge_index, W_l, b_l, W_r):
    src = edge_index[0]
    dst = edge_index[1]
    xpad = jnp.concatenate([x, jnp.ones((N, ROWW - D), jnp.float32)], axis=1)
    zeros = jnp.zeros((ZPT, ROWW), jnp.float32)
    partial = _sc_aggregate(xpad, src, dst, zeros)
    partial = partial.reshape(NC, N, ROWW)
    return _tc_combine(partial, x, W_l.T, W_r.T, b_l.reshape(1, D))


# speedup vs baseline: 9.0408x; 1.0841x over previous
import functools

import jax
import jax.numpy as jnp
from jax import lax
from jax.experimental import pallas as pl
from jax.experimental.pallas import tpu as pltpu
from jax.experimental.pallas import tpu_sc as plsc

N = 10000
NPAD = 10240
E = 320000
D = 128
ROWW = D + 16
NC, NS = 2, 16
NW = NC * NS
CH = 256
NCHUNKS = E // CH
SLOTS = 40
ZPT = NPAD // NS
DPT = N // NS


def _sc_body(xpad_hbm, src_hbm, dst_hbm, zeros_hbm, out_hbm,
             sidx0, didx0, sidx1, didx1, rows, acc,
             si0, si1, sg0, sz):
    c = lax.axis_index("c")
    s = lax.axis_index("s")
    wid = c * NS + s

    zslice = acc.at[pl.ds(pl.multiple_of(s * ZPT, 8), ZPT)]
    pltpu.async_copy(zeros_hbm, zslice, sz)

    def off(t):
        return jnp.minimum(wid + t * NW, NCHUNKS - 1) * CH

    def idx_start(t, sbuf, dbuf, sem):
        o = off(t)
        pltpu.async_copy(src_hbm.at[pl.ds(o, CH)], sbuf, sem)
        pltpu.async_copy(dst_hbm.at[pl.ds(o, CH)], dbuf, sem)

    def idx_wait(sbuf, dbuf, sem):
        pltpu.make_async_copy(src_hbm.at[pl.ds(0, CH)], sbuf, sem).wait()
        pltpu.make_async_copy(dst_hbm.at[pl.ds(0, CH)], dbuf, sem).wait()

    def gath(sbuf, rbuf, sem):
        pltpu.async_copy(xpad_hbm.at[sbuf], rbuf, sem)

    def gwait(sbuf, rbuf, sem):
        pltpu.make_async_copy(xpad_hbm.at[sbuf], rbuf, sem).wait()

    def scat(t, dbuf):
        @pl.when(wid + t * NW < NCHUNKS)
        def _():
            pltpu.sync_copy(rows, acc.at[dbuf], add=True)

    idx_start(0, sidx0, didx0, si0)
    idx_start(1, sidx1, didx1, si1)
    idx_wait(sidx0, didx0, si0)
    gath(sidx0, rows, sg0)
    pltpu.make_async_copy(zeros_hbm, zslice, sz).wait()
    plsc.subcore_barrier()

    def body(i, carry):
        a = 2 * i
        b = a + 1
        gwait(sidx0, rows, sg0)
        scat(a, didx0)
        idx_start(a + 2, sidx0, didx0, si0)
        idx_wait(sidx1, didx1, si1)
        gath(sidx1, rows, sg0)
        gwait(sidx1, rows, sg0)
        scat(b, didx1)
        idx_start(b + 2, sidx1, didx1, si1)
        idx_wait(sidx0, didx0, si0)
        gath(sidx0, rows, sg0)
        return carry

    lax.fori_loop(0, SLOTS // 2, body, 0)
    idx_wait(sidx1, didx1, si1)
    gwait(sidx0, rows, sg0)

    plsc.subcore_barrier()
    pltpu.sync_copy(acc.at[pl.ds(s * DPT, DPT)],
                    out_hbm.at[pl.ds(c * N + s * DPT, DPT)])


_sc_aggregate = functools.partial(
    pl.kernel,
    out_type=jax.ShapeDtypeStruct((NC * N, ROWW), jnp.float32),
    mesh=plsc.VectorSubcoreMesh(core_axis_name="c", subcore_axis_name="s",
                                num_cores=NC, num_subcores=NS),
    scratch_types=[
        pltpu.VMEM((CH,), jnp.int32),
        pltpu.VMEM((CH,), jnp.int32),
        pltpu.VMEM((CH,), jnp.int32),
        pltpu.VMEM((CH,), jnp.int32),
        pltpu.VMEM((CH, ROWW), jnp.float32),
        pltpu.VMEM_SHARED((NPAD, ROWW), jnp.float32),
        pltpu.SemaphoreType.DMA,
        pltpu.SemaphoreType.DMA,
        pltpu.SemaphoreType.DMA,
        pltpu.SemaphoreType.DMA,
    ],
    compiler_params=pltpu.CompilerParams(use_tc_tiling_on_sc=False),
)(_sc_body)


def _tc_body(p_ref, x_ref, wl_ref, wr_ref, b_ref, o_ref):
    p = p_ref[0] + p_ref[1]
    cnt = p[:, D:D + 1]
    mean = p[:, :D] / jnp.maximum(cnt, 1.0)
    o_ref[...] = (
        jnp.dot(mean, wl_ref[...], preferred_element_type=jnp.float32)
        + jnp.dot(x_ref[...], wr_ref[...], preferred_element_type=jnp.float32)
        + b_ref[...]
    )


def _tc_combine(partial, x, wl_t, wr_t, b2):
    R = 1000
    grid = (N // R,)
    return pl.pallas_call(
        _tc_body,
        grid=grid,
        in_specs=[
            pl.BlockSpec((NC, R, ROWW), lambda i: (0, i, 0)),
            pl.BlockSpec((R, D), lambda i: (i, 0)),
            pl.BlockSpec((D, D), lambda i: (0, 0)),
            pl.BlockSpec((D, D), lambda i: (0, 0)),
            pl.BlockSpec((1, D), lambda i: (0, 0)),
        ],
        out_specs=pl.BlockSpec((R, D), lambda i: (i, 0)),
        out_shape=jax.ShapeDtypeStruct((N, D), jnp.float32),
    )(partial, x, wl_t, wr_t, b2)


def kernel(x, edge_index, W_l, b_l, W_r):
    src = edge_index[0]
    dst = edge_index[1]
    xpad = jnp.concatenate([x, jnp.ones((N, ROWW - D), jnp.float32)], axis=1)
    zeros = jnp.zeros((ZPT, ROWW), jnp.float32)
    partial = _sc_aggregate(xpad, src, dst, zeros)
    partial = partial.reshape(NC, N, ROWW)
    return _tc_combine(partial, x, W_l.T, W_r.T, b_l.reshape(1, D))


# speedup vs baseline: 9.3991x; 1.0396x over previous
import functools

import jax
import jax.numpy as jnp
from jax import lax
from jax.experimental import pallas as pl
from jax.experimental.pallas import tpu as pltpu
from jax.experimental.pallas import tpu_sc as plsc

N = 10000
NPAD = 10240
E = 320000
D = 128
ROWW = D + 16
NC, NS = 2, 16
NW = NC * NS
CH = 256
NCHUNKS = E // CH
SLOTS = 40
ZPT = NPAD // NS
DPT = N // NS


def _sc_body(xpad_hbm, ei_hbm, zeros_hbm, out_hbm,
             eidx0, eidx1, rows, acc,
             si0, si1, sg0, sz):
    c = lax.axis_index("c")
    s = lax.axis_index("s")
    wid = c * NS + s

    zslice = acc.at[pl.ds(pl.multiple_of(s * ZPT, 8), ZPT)]
    pltpu.async_copy(zeros_hbm, zslice, sz)

    def off(t):
        return jnp.minimum(wid + t * NW, NCHUNKS - 1) * CH

    def idx_start(t, ebuf, sem):
        pltpu.async_copy(ei_hbm.at[:, pl.ds(off(t), CH)], ebuf, sem)

    def idx_wait(ebuf, sem):
        pltpu.make_async_copy(ei_hbm.at[:, pl.ds(0, CH)], ebuf, sem).wait()

    def gath(sbuf, rbuf, sem):
        pltpu.async_copy(xpad_hbm.at[sbuf], rbuf, sem)

    def gwait(sbuf, rbuf, sem):
        pltpu.make_async_copy(xpad_hbm.at[sbuf], rbuf, sem).wait()

    def scat(t, ebuf):
        @pl.when(wid + t * NW < NCHUNKS)
        def _():
            pltpu.sync_copy(rows, acc.at[ebuf.at[1]], add=True)

    idx_start(0, eidx0, si0)
    idx_start(1, eidx1, si1)
    idx_wait(eidx0, si0)
    gath(eidx0.at[0], rows, sg0)
    pltpu.make_async_copy(zeros_hbm, zslice, sz).wait()
    plsc.subcore_barrier()

    def body(i, carry):
        a = 2 * i
        b = a + 1
        gwait(eidx0.at[0], rows, sg0)
        scat(a, eidx0)
        idx_start(a + 2, eidx0, si0)
        idx_wait(eidx1, si1)
        gath(eidx1.at[0], rows, sg0)
        gwait(eidx1.at[0], rows, sg0)
        scat(b, eidx1)
        idx_start(b + 2, eidx1, si1)
        idx_wait(eidx0, si0)
        gath(eidx0.at[0], rows, sg0)
        return carry

    lax.fori_loop(0, SLOTS // 2, body, 0)
    idx_wait(eidx1, si1)
    gwait(eidx0.at[0], rows, sg0)

    plsc.subcore_barrier()
    pltpu.sync_copy(acc.at[pl.ds(s * DPT, DPT)],
                    out_hbm.at[pl.ds(c * N + s * DPT, DPT)])


_sc_aggregate = functools.partial(
    pl.kernel,
    out_type=jax.ShapeDtypeStruct((NC * N, ROWW), jnp.float32),
    mesh=plsc.VectorSubcoreMesh(core_axis_name="c", subcore_axis_name="s",
                                num_cores=NC, num_subcores=NS),
    scratch_types=[
        pltpu.VMEM((2, CH), jnp.int32),
        pltpu.VMEM((2, CH), jnp.int32),
        pltpu.VMEM((CH, ROWW), jnp.float32),
        pltpu.VMEM_SHARED((NPAD, ROWW), jnp.float32),
        pltpu.SemaphoreType.DMA,
        pltpu.SemaphoreType.DMA,
        pltpu.SemaphoreType.DMA,
        pltpu.SemaphoreType.DMA,
    ],
    compiler_params=pltpu.CompilerParams(use_tc_tiling_on_sc=False),
)(_sc_body)


def _tc_body(p_ref, x_ref, wl_ref, wr_ref, b_ref, o_ref):
    p = p_ref[0] + p_ref[1]
    cnt = p[:, D:D + 1]
    mean = p[:, :D] / jnp.maximum(cnt, 1.0)
    dn = (((1,), (1,)), ((), ()))
    o_ref[...] = (
        lax.dot_general(mean, wl_ref[...], dn,
                        preferred_element_type=jnp.float32)
        + lax.dot_general(x_ref[...], wr_ref[...], dn,
                          preferred_element_type=jnp.float32)
        + b_ref[...]
    )


def _tc_combine(partial, x, wl_t, wr_t, b2):
    R = 1000
    grid = (N // R,)
    return pl.pallas_call(
        _tc_body,
        grid=grid,
        in_specs=[
            pl.BlockSpec((NC, R, ROWW), lambda i: (0, i, 0)),
            pl.BlockSpec((R, D), lambda i: (i, 0)),
            pl.BlockSpec((D, D), lambda i: (0, 0)),
            pl.BlockSpec((D, D), lambda i: (0, 0)),
            pl.BlockSpec((1, D), lambda i: (0, 0)),
        ],
        out_specs=pl.BlockSpec((R, D), lambda i: (i, 0)),
        out_shape=jax.ShapeDtypeStruct((N, D), jnp.float32),
    )(partial, x, wl_t, wr_t, b2)


def kernel(x, edge_index, W_l, b_l, W_r):
    xpad = jnp.concatenate([x, jnp.ones((N, ROWW - D), jnp.float32)], axis=1)
    zeros = jnp.zeros((ZPT, ROWW), jnp.float32)
    partial = _sc_aggregate(xpad, edge_index, zeros)
    partial = partial.reshape(NC, N, ROWW)
    return _tc_combine(partial, x, W_l, W_r, b_l.reshape(1, D))
